# baseline (speedup 1.0000x reference)
import functools

import jax
import jax.numpy as jnp
from jax import lax
from jax.experimental import pallas as pl
from jax.experimental.pallas import tpu as pltpu
from jax.experimental.pallas import tpu_sc as plsc

B, L, D = 4096, 50, 128
LP = 56
NC, NS = 2, 16
NW = NC * NS
BPW = B // NW

SELU_ALPHA = 1.6732632423543772
SELU_SCALE = 1.0507009873554805


def _sc_cbow_body(idx_hbm, table_hbm, out_hbm, idx_v, rows_v, acc_v, sem):
    wid = lax.axis_index("s") * NC + lax.axis_index("c")
    base = wid * BPW
    pltpu.sync_copy(idx_hbm.at[pl.ds(base, BPW), :], idx_v)

    def row(i, _):
        cp = pltpu.async_copy(table_hbm.at[idx_v.at[i]], rows_v, sem)
        cp.wait()
        for d in range(D // 16):
            sl = pl.ds(d * 16, 16)
            v = rows_v[0, sl]
            for l in range(1, LP):
                v = v + rows_v[l, sl]
            acc_v[i, sl] = v
        return 0

    lax.fori_loop(0, BPW, row, 0)
    pltpu.sync_copy(acc_v, out_hbm.at[pl.ds(base, BPW), :])


@functools.partial(jax.jit, static_argnums=())
def _sc_cbow(idx_pad, table):
    mesh = plsc.VectorSubcoreMesh(core_axis_name="c", subcore_axis_name="s")
    return pl.kernel(
        _sc_cbow_body,
        out_type=jax.ShapeDtypeStruct((B, D), jnp.float32),
        mesh=mesh,
        scratch_types=[
            pltpu.VMEM((BPW, LP), jnp.int32),
            pltpu.VMEM((LP, D), jnp.float32),
            pltpu.VMEM((BPW, D), jnp.float32),
            pltpu.SemaphoreType.DMA,
        ],
    )(idx_pad, table)


def _tc_proj_body(y_ref, w_ref, b_ref, o_ref):
    y = y_ref[...]
    s = jnp.where(y > 0, y, SELU_ALPHA * (jnp.exp(y) - 1.0)) * SELU_SCALE
    o_ref[...] = (
        lax.dot_general(s, w_ref[...], (((1,), (1,)), ((), ())),
                        preferred_element_type=jnp.float32)
        + b_ref[...]
    )


def _tc_proj(y, W, b2d):
    nblk = 8
    blk = B // nblk
    return pl.pallas_call(
        _tc_proj_body,
        grid=(nblk,),
        in_specs=[
            pl.BlockSpec((blk, D), lambda i: (i, 0)),
            pl.BlockSpec((D, D), lambda i: (0, 0)),
            pl.BlockSpec((1, D), lambda i: (0, 0)),
        ],
        out_specs=pl.BlockSpec((blk, D), lambda i: (i, 0)),
        out_shape=jax.ShapeDtypeStruct((B, D), jnp.float32),
    )(y, W, b2d)


def kernel(input_text, table, W, b):
    idx = input_text.reshape(B, L).astype(jnp.int32)
    idx_pad = jnp.pad(idx, ((0, 0), (0, LP - L)))
    y = _sc_cbow(idx_pad, table)
    return _tc_proj(y, W, b.reshape(1, D))
